# Initial kernel scaffold; baseline (speedup 1.0000x reference)
import jax
import jax.numpy as jnp
from jax.experimental import pallas as pl


def kernel(x, adj_indices, adj_values, mlp_W0, mlp_b0, mlp_W1, mlp_b1, d0_Wlin, d0_blin, d0_Wih, d0_Whh, d0_bih, d0_bhh, d0_gamma, d0_beta, d1_Wlin, d1_blin, d1_Wih, d1_Whh, d1_bih, d1_bhh, d1_gamma, d1_beta):
    raise NotImplementedError("write your pallas kernel here")


# speedup vs baseline: 2.7009x; 2.7009x over previous
import functools

import jax
import jax.numpy as jnp
from jax import lax
from jax.experimental import pallas as pl
from jax.experimental.pallas import tpu as pltpu
from jax.experimental.pallas import tpu_sc as plsc

NC = 2
NS = 16
NW = NC * NS
LANES = 16
CHUNK = 128


def _sc_hops(x, srcp, dstp, valp, n_nodes, d, k_hops, n_chunks):
    stripe = (n_nodes // NS) // 8 * 8
    tail = n_nodes - NS * stripe
    zrows = 16

    mesh = plsc.VectorSubcoreMesh(core_axis_name="c", subcore_axis_name="s")

    @functools.partial(
        pl.kernel,
        mesh=mesh,
        out_type=jax.ShapeDtypeStruct((k_hops, NC, n_nodes, d), jnp.float32),
        scratch_types=[
            pltpu.VMEM_SHARED((n_nodes, d), jnp.float32),
            pltpu.VMEM((CHUNK,), jnp.int32),
            pltpu.VMEM((CHUNK,), jnp.int32),
            pltpu.VMEM((CHUNK,), jnp.float32),
            pltpu.VMEM((CHUNK, 128), jnp.float32),
            pltpu.VMEM((zrows, 128), jnp.float32),
            pltpu.SemaphoreType.DMA,
        ],
    )
    def body(x_hbm, src_hbm, dst_hbm, val_hbm, out_hbm,
             acc, sidx, didx, vals, rows, zbuf, sem):
        cid = lax.axis_index("c")
        sid = lax.axis_index("s")
        wid = cid * NS + sid
        row0 = sid * stripe

        def _zb(i, _):
            for d8 in range(d // LANES):
                zbuf[i, pl.ds(d8 * LANES, LANES)] = jnp.zeros((LANES,), jnp.float32)
            return 0
        lax.fori_loop(0, zrows, _zb, 0)

        for k in range(k_hops):
            def _zero(j, _):
                pltpu.sync_copy(zbuf, acc.at[pl.ds(row0 + j * zrows, zrows)])
                return 0
            lax.fori_loop(0, stripe // zrows, _zero, 0)
            @pl.when(sid == 0)
            def _zero_tail():
                def _zt(j, _):
                    pltpu.sync_copy(
                        zbuf, acc.at[pl.ds(NS * stripe + j * zrows, zrows)])
                    return 0
                lax.fori_loop(0, tail // zrows, _zt, 0)
            plsc.subcore_barrier()

            def _chunk(j, _):
                pltpu.sync_copy(src_hbm.at[k, wid, j], sidx)
                pltpu.sync_copy(dst_hbm.at[k, wid, j], didx)
                pltpu.sync_copy(val_hbm.at[k, wid, j], vals)
                pltpu.async_copy(x_hbm.at[sidx], rows, sem).wait()

                def _scale(g, _):
                    v16 = vals[pl.ds(g * LANES, LANES)]
                    for e in range(LANES):
                        s = v16[e]
                        i = g * LANES + e
                        for d8 in range(d // LANES):
                            sl = pl.ds(d8 * LANES, LANES)
                            rows[i, sl] = rows[i, sl] * s
                    return 0
                lax.fori_loop(0, CHUNK // LANES, _scale, 0)

                pltpu.sync_copy(rows, acc.at[didx], add=True)
                return 0
            lax.fori_loop(0, n_chunks, _chunk, 0)
            plsc.subcore_barrier()

            pltpu.sync_copy(acc.at[pl.ds(row0, stripe)],
                            out_hbm.at[k, cid, pl.ds(row0, stripe)])
            @pl.when(sid == 0)
            def _drain_tail():
                pltpu.sync_copy(acc.at[pl.ds(NS * stripe, tail)],
                                out_hbm.at[k, cid, pl.ds(NS * stripe, tail)])
            plsc.subcore_barrier()

    return body(x, srcp, dstp, valp)


def _mlp_tc(x, w0, b0, w1, b1, bn):
    n, d = x.shape

    def body(x_ref, w0_ref, b0_ref, w1_ref, b1_ref, o_ref):
        h = jnp.dot(x_ref[...], w0_ref[...], preferred_element_type=jnp.float32)
        h = jnp.maximum(h + b0_ref[...], 0.0)
        o = jnp.dot(h, w1_ref[...], preferred_element_type=jnp.float32)
        o_ref[...] = o + b1_ref[...]

    return pl.pallas_call(
        body,
        grid=(n // bn,),
        in_specs=[
            pl.BlockSpec((bn, d), lambda i: (i, 0)),
            pl.BlockSpec((d, d), lambda i: (0, 0)),
            pl.BlockSpec((1, d), lambda i: (0, 0)),
            pl.BlockSpec((d, d), lambda i: (0, 0)),
            pl.BlockSpec((1, d), lambda i: (0, 0)),
        ],
        out_specs=pl.BlockSpec((bn, d), lambda i: (i, 0)),
        out_shape=jax.ShapeDtypeStruct((n, d), jnp.float32),
    )(x, w0, b0.reshape(1, d), w1, b1.reshape(1, d))


def _layer_tc(partials, wlin, blin, wih, whh, bih, bhh, gamma, beta, bn):
    k_hops, _, n, d = partials.shape

    def body(p_ref, wlin_ref, blin_ref, wih_ref, whh_ref, bih_ref, bhh_ref,
             g_ref, bta_ref, o_ref):
        h = jnp.zeros((bn, d), jnp.float32)
        acc = jnp.zeros((bn, d), jnp.float32)
        for k in range(k_hops):
            agg = p_ref[k, 0] + p_ref[k, 1]
            hs = jnp.dot(agg, wlin_ref[...], preferred_element_type=jnp.float32)
            hs = jnp.maximum(hs + blin_ref[...], 0.0)
            gi = lax.dot_general(hs, wih_ref[...], (((1,), (1,)), ((), ())),
                                 preferred_element_type=jnp.float32) + bih_ref[...]
            gh = lax.dot_general(h, whh_ref[...], (((1,), (1,)), ((), ())),
                                 preferred_element_type=jnp.float32) + bhh_ref[...]
            r = jax.nn.sigmoid(gi[:, :d] + gh[:, :d])
            z = jax.nn.sigmoid(gi[:, d:2 * d] + gh[:, d:2 * d])
            nn = jnp.tanh(gi[:, 2 * d:] + r * gh[:, 2 * d:])
            h = (1.0 - z) * nn + z * h
            acc = acc + h
        mu = jnp.mean(acc, axis=-1, keepdims=True)
        var = jnp.mean((acc - mu) ** 2, axis=-1, keepdims=True)
        o_ref[...] = (acc - mu) * lax.rsqrt(var + 1e-5) * g_ref[...] + bta_ref[...]

    return pl.pallas_call(
        body,
        grid=(n // bn,),
        in_specs=[
            pl.BlockSpec((k_hops, NC, bn, d), lambda i: (0, 0, i, 0)),
            pl.BlockSpec((d, d), lambda i: (0, 0)),
            pl.BlockSpec((1, d), lambda i: (0, 0)),
            pl.BlockSpec((3 * d, d), lambda i: (0, 0)),
            pl.BlockSpec((3 * d, d), lambda i: (0, 0)),
            pl.BlockSpec((1, 3 * d), lambda i: (0, 0)),
            pl.BlockSpec((1, 3 * d), lambda i: (0, 0)),
            pl.BlockSpec((1, d), lambda i: (0, 0)),
            pl.BlockSpec((1, d), lambda i: (0, 0)),
        ],
        out_specs=pl.BlockSpec((bn, d), lambda i: (i, 0)),
        out_shape=jax.ShapeDtypeStruct((n, d), jnp.float32),
    )(partials, wlin, blin.reshape(1, d), wih, whh, bih.reshape(1, 3 * d),
      bhh.reshape(1, 3 * d), gamma.reshape(1, d), beta.reshape(1, d))


def _prep_edges(adj_indices, adj_values):
    k_hops, _, e = adj_indices.shape
    epw = e // NW
    n_chunks = -(-epw // CHUNK)
    pad = n_chunks * CHUNK - epw
    dst = adj_indices[:, 0].reshape(k_hops, NW, epw)
    src = adj_indices[:, 1].reshape(k_hops, NW, epw)
    val = adj_values.reshape(k_hops, NW, epw)
    if pad:
        dst = jnp.pad(dst, ((0, 0), (0, 0), (0, pad)))
        src = jnp.pad(src, ((0, 0), (0, 0), (0, pad)))
        val = jnp.pad(val, ((0, 0), (0, 0), (0, pad)))
    shape = (k_hops, NW, n_chunks, CHUNK)
    return (src.reshape(shape), dst.reshape(shape), val.reshape(shape),
            n_chunks)


def kernel(x, adj_indices, adj_values, mlp_W0, mlp_b0, mlp_W1, mlp_b1,
           d0_Wlin, d0_blin, d0_Wih, d0_Whh, d0_bih, d0_bhh, d0_gamma, d0_beta,
           d1_Wlin, d1_blin, d1_Wih, d1_Whh, d1_bih, d1_bhh, d1_gamma, d1_beta):
    n, d = x.shape
    k_hops = adj_indices.shape[0]
    srcp, dstp, valp, n_chunks = _prep_edges(adj_indices, adj_values)

    trans = _mlp_tc(x, mlp_W0, mlp_b0, mlp_W1, mlp_b1, bn=1000)

    h = trans
    for wlin, blin, wih, whh, bih, bhh, gamma, beta in (
        (d0_Wlin, d0_blin, d0_Wih, d0_Whh, d0_bih, d0_bhh, d0_gamma, d0_beta),
        (d1_Wlin, d1_blin, d1_Wih, d1_Whh, d1_bih, d1_bhh, d1_gamma, d1_beta),
    ):
        partials = _sc_hops(h, srcp, dstp, valp, n, d, k_hops, n_chunks)
        h = _layer_tc(partials, wlin, blin, wih, whh, bih, bhh, gamma, beta,
                      bn=1000)
    return (h, trans)
